# baseline (speedup 1.0000x reference)
import functools

import jax
import jax.numpy as jnp
from jax.experimental import pallas as pl
from jax.experimental.pallas import tpu as pltpu

_E = 8
_K = 2
_BT = 8192


def _router_kernel(n_tokens, x_ref, w_ref, tkwt_ref, tkit_ref, aux_ref,
                   psum_ref, cnt_ref):
    i = pl.program_id(0)
    n = pl.num_programs(0)

    @pl.when(i == 0)
    def _init():
        psum_ref[...] = jnp.zeros_like(psum_ref)
        cnt_ref[...] = jnp.zeros_like(cnt_ref)

    x = x_ref[...]
    w = w_ref[...]
    logits = jax.lax.dot_general(
        x, w, (((1,), (1,)), ((), ())),
        preferred_element_type=jnp.float32)
    lt = logits.T

    e_iota = jax.lax.broadcasted_iota(jnp.int32, lt.shape, 0)
    m1 = jnp.max(lt, axis=0, keepdims=True)
    i1 = jnp.min(jnp.where(lt == m1, e_iota, _E), axis=0,
                 keepdims=True)
    masked = jnp.where(e_iota == i1, -jnp.inf, lt)
    m2 = jnp.max(masked, axis=0, keepdims=True)
    i2 = jnp.min(jnp.where(masked == m2, e_iota, _E), axis=0,
                 keepdims=True)

    d = jnp.exp(m2 - m1)
    w1 = 1.0 / (1.0 + d)
    tkwt_ref[...] = jnp.concatenate([w1, 1.0 - w1], axis=0)
    tkit_ref[...] = jnp.concatenate([i1, i2], axis=0)

    p = jnp.exp(lt - m1)
    probs = p / jnp.sum(p, axis=0, keepdims=True)
    psum_ref[...] += probs
    onehot = ((e_iota == i1).astype(jnp.float32)
              + (e_iota == i2).astype(jnp.float32))
    cnt_ref[...] += onehot

    @pl.when(i == n - 1)
    def _finish():
        c = jnp.sum(cnt_ref[...], axis=1, keepdims=True)
        s = jnp.sum(psum_ref[...], axis=1, keepdims=True)
        aux_ref[0, 0] = (_E * jnp.sum(c * s)
                         / (n_tokens * _K * n_tokens))


def kernel(routing_features, W):
    n_tokens, d_model = routing_features.shape
    grid = n_tokens // _BT

    body = functools.partial(_router_kernel, float(n_tokens))

    tkwt, tkit, aux = pl.pallas_call(
        body,
        grid=(grid,),
        in_specs=[
            pl.BlockSpec((_BT, d_model), lambda i: (i, 0)),
            pl.BlockSpec((_E, d_model), lambda i: (0, 0)),
        ],
        out_specs=[
            pl.BlockSpec((_K, _BT), lambda i: (0, i)),
            pl.BlockSpec((_K, _BT), lambda i: (0, i)),
            pl.BlockSpec(memory_space=pltpu.SMEM),
        ],
        out_shape=[
            jax.ShapeDtypeStruct((_K, n_tokens), jnp.float32),
            jax.ShapeDtypeStruct((_K, n_tokens), jnp.int32),
            jax.ShapeDtypeStruct((1, 1), jnp.float32),
        ],
        scratch_shapes=[
            pltpu.VMEM((_E, _BT), jnp.float32),
            pltpu.VMEM((_E, _BT), jnp.float32),
        ],
    )(routing_features, W)
    return tkwt.T, tkit.T, aux[0, 0]


# speedup vs baseline: 1.0903x; 1.0903x over previous
import functools

import jax
import jax.numpy as jnp
from jax.experimental import pallas as pl
from jax.experimental.pallas import tpu as pltpu

_E = 8
_K = 2
_BT = 4096


def _router_kernel(n_tokens, x0_ref, x1_ref, w_ref, tkwt_ref, tkit_ref,
                   aux_ref, psum_ref, cnt_ref):
    i = pl.program_id(0)
    n = pl.num_programs(0)

    @pl.when(i == 0)
    def _init():
        psum_ref[...] = jnp.zeros_like(psum_ref)
        cnt_ref[...] = jnp.zeros_like(cnt_ref)

    w = w_ref[...]
    dn = (((1,), (1,)), ((), ()))
    lt0 = jax.lax.dot_general(
        x0_ref[...], w, dn, preferred_element_type=jnp.float32).T
    lt1 = jax.lax.dot_general(
        x1_ref[...], w, dn, preferred_element_type=jnp.float32).T
    lt = jnp.concatenate([lt0, lt1], axis=1)

    e_iota = jax.lax.broadcasted_iota(jnp.int32, lt.shape, 0)
    m1 = jnp.max(lt, axis=0, keepdims=True)
    i1 = jnp.min(jnp.where(lt == m1, e_iota, _E), axis=0,
                 keepdims=True)
    masked = jnp.where(e_iota == i1, -jnp.inf, lt)
    m2 = jnp.max(masked, axis=0, keepdims=True)
    i2 = jnp.min(jnp.where(masked == m2, e_iota, _E), axis=0,
                 keepdims=True)

    d = jnp.exp(m2 - m1)
    w1 = 1.0 / (1.0 + d)
    tkwt_ref[...] = jnp.concatenate([w1, 1.0 - w1], axis=0)
    tkit_ref[...] = jnp.concatenate([i1, i2], axis=0)

    p = jnp.exp(lt - m1)
    probs = p / jnp.sum(p, axis=0, keepdims=True)
    psum_ref[...] += probs
    onehot = ((e_iota == i1).astype(jnp.float32)
              + (e_iota == i2).astype(jnp.float32))
    cnt_ref[...] += onehot

    @pl.when(i == n - 1)
    def _finish():
        c = jnp.sum(cnt_ref[...], axis=1, keepdims=True)
        s = jnp.sum(psum_ref[...], axis=1, keepdims=True)
        aux_ref[0, 0] = (_E * jnp.sum(c * s)
                         / (n_tokens * _K * n_tokens))


def kernel(routing_features, W):
    n_tokens, d_model = routing_features.shape
    grid = n_tokens // _BT

    body = functools.partial(_router_kernel, float(n_tokens))

    tkwt, tkit, aux = pl.pallas_call(
        body,
        grid=(grid,),
        in_specs=[
            pl.BlockSpec((_BT // 2, d_model), lambda i: (2 * i, 0)),
            pl.BlockSpec((_BT // 2, d_model), lambda i: (2 * i + 1, 0)),
            pl.BlockSpec((_E, d_model), lambda i: (0, 0)),
        ],
        out_specs=[
            pl.BlockSpec((_K, _BT), lambda i: (0, i)),
            pl.BlockSpec((_K, _BT), lambda i: (0, i)),
            pl.BlockSpec(memory_space=pltpu.SMEM),
        ],
        out_shape=[
            jax.ShapeDtypeStruct((_K, n_tokens), jnp.float32),
            jax.ShapeDtypeStruct((_K, n_tokens), jnp.int32),
            jax.ShapeDtypeStruct((1, 1), jnp.float32),
        ],
        scratch_shapes=[
            pltpu.VMEM((_E, _BT), jnp.float32),
            pltpu.VMEM((_E, _BT), jnp.float32),
        ],
    )(routing_features, routing_features, W)
    return tkwt.T, tkit.T, aux[0, 0]
